# baseline (speedup 1.0000x reference)
import functools

import jax
import jax.numpy as jnp
from jax import lax
from jax.experimental import pallas as pl
from jax.experimental.pallas import tpu as pltpu
from jax.experimental.pallas import tpu_sc as plsc

NC = 2
NS = 16
NW = NC * NS
L = 16
K = 96
ZR = 16
NBUF = 3
CH = 9


def _sc_mesh():
    return plsc.VectorSubcoreMesh(
        core_axis_name="c", subcore_axis_name="s",
        num_cores=NC, num_subcores=NS)


def _make_deg_kernel(n_pad, nblk):
    rpt = n_pad // NS

    def body(dst_hbm, out_hbm, idx_v, ones_v, zero_v, acc_sh, sem):
        c = lax.axis_index("c")
        s = lax.axis_index("s")
        wid = c * NS + s
        for i in range(K):
            ones_v[i] = jnp.full((L,), 1.0, jnp.float32)
        for i in range(ZR):
            zero_v[i] = jnp.zeros((L,), jnp.float32)

        @pl.loop(0, rpt // ZR)
        def _zero(r):
            pltpu.sync_copy(zero_v, acc_sh.at[pl.ds(s * rpt + r * ZR, ZR)])

        plsc.subcore_barrier()
        pltpu.sync_copy(dst_hbm.at[wid], idx_v)

        @pl.loop(0, nblk // CH)
        def _scatter(ci):
            for off in range(CH):
                pltpu.async_copy(ones_v, acc_sh.at[idx_v.at[ci, off]], sem,
                                 add=True)

        @pl.loop(0, nblk // CH)
        def _drain(ci):
            for off in range(CH):
                pltpu.make_async_copy(ones_v, acc_sh.at[idx_v.at[0, 0]],
                                      sem).wait()

        plsc.subcore_barrier()
        pltpu.sync_copy(acc_sh.at[pl.ds(s * rpt, rpt)],
                        out_hbm.at[c, pl.ds(s * rpt, rpt)])

    return pl.kernel(
        body,
        out_type=jax.ShapeDtypeStruct((NC, n_pad, L), jnp.float32),
        mesh=_sc_mesh(),
        scratch_types=[
            pltpu.VMEM((nblk // CH, CH, K), jnp.int32),
            pltpu.VMEM((K, L), jnp.float32),
            pltpu.VMEM((ZR, L), jnp.float32),
            pltpu.VMEM_SHARED((n_pad, L), jnp.float32),
            pltpu.SemaphoreType.DMA,
        ])


def _make_agg_kernel(n_pad, nblk, d):
    rpt = n_pad // NS
    nch = nblk // CH

    def body(y_hbm, src_hbm, dst_hbm, out_hbm,
             sidx_v, didx_v, rows_v, acc_sh, gsem, ssem, isem):
        c = lax.axis_index("c")
        s = lax.axis_index("s")
        wid = c * NS + s
        for i in range(ZR):
            for j in range(d // L):
                rows_v[0, i, pl.ds(j * L, L)] = jnp.zeros((L,), jnp.float32)

        @pl.loop(0, rpt // ZR)
        def _zero(r):
            pltpu.sync_copy(rows_v.at[0, pl.ds(0, ZR)],
                            acc_sh.at[pl.ds(s * rpt + r * ZR, ZR)])

        plsc.subcore_barrier()

        idesc = [None] * nch
        gdesc = [None] * nblk
        sdesc = [None] * nblk

        def idx_load(ci):
            ib = ci % 2
            idesc[ci] = (
                pltpu.async_copy(src_hbm.at[wid, ci], sidx_v.at[ib],
                                 isem.at[ib]),
                pltpu.async_copy(dst_hbm.at[wid, ci], didx_v.at[ib],
                                 isem.at[ib]))

        def idx_wait(ci):
            idesc[ci][0].wait()
            idesc[ci][1].wait()

        def gather(blk):
            ci, off = divmod(blk, CH)
            gdesc[blk] = pltpu.async_copy(
                y_hbm.at[sidx_v.at[ci % 2, off]], rows_v.at[blk % NBUF],
                gsem.at[blk % NBUF])

        def scat(blk):
            ci, off = divmod(blk, CH)
            sdesc[blk] = pltpu.async_copy(
                rows_v.at[blk % NBUF], acc_sh.at[didx_v.at[ci % 2, off]],
                ssem.at[blk % NBUF], add=True)

        idx_load(0)
        idx_wait(0)
        if nch > 1:
            idx_load(1)
        waited = {0}
        for blk in range(min(NBUF, nblk)):
            gather(blk)
        for blk in range(nblk):
            gdesc[blk].wait()
            scat(blk)
            nxt = blk + NBUF
            if nxt < nblk:
                sdesc[blk].wait()
                m = blk // CH
                if blk % CH == 1 and m >= 1 and m + 1 < nch:
                    idx_load(m + 1)
                nci = nxt // CH
                if nci not in waited:
                    idx_wait(nci)
                    waited.add(nci)
                gather(nxt)
        for blk in range(max(0, nblk - NBUF), nblk):
            sdesc[blk].wait()
        plsc.subcore_barrier()
        pltpu.sync_copy(acc_sh.at[pl.ds(s * rpt, rpt)],
                        out_hbm.at[c, pl.ds(s * rpt, rpt)])

    return pl.kernel(
        body,
        out_type=jax.ShapeDtypeStruct((NC, n_pad, d), jnp.float32),
        mesh=_sc_mesh(),
        scratch_types=[
            pltpu.VMEM((2, CH, K), jnp.int32),
            pltpu.VMEM((2, CH, K), jnp.int32),
            pltpu.VMEM((NBUF, K, d), jnp.float32),
            pltpu.VMEM_SHARED((n_pad, d), jnp.float32),
            pltpu.SemaphoreType.DMA((NBUF,)),
            pltpu.SemaphoreType.DMA((NBUF,)),
            pltpu.SemaphoreType.DMA((2,)),
        ])


def _dinv(deg_blk):
    return lax.rsqrt(deg_blk[0, :, 0:1] + deg_blk[1, :, 0:1])


def _pick_bn(n):
    for bn in (2048, 2000, 1024, 1000, 512, 500, 256, 250, 128, 125, 64, 50,
               40, 32, 25, 20, 16, 10, 8, 5, 4, 2, 1):
        if n % bn == 0:
            return bn
    return n


def _tc_matmul(x, w, b, bn):
    n, d = x.shape

    def body(x_ref, w_ref, b_ref, y_ref):
        y_ref[...] = jnp.dot(x_ref[...], w_ref[...],
                             preferred_element_type=jnp.float32) + b_ref[...]

    return pl.pallas_call(
        body,
        grid=(n // bn,),
        in_specs=[
            pl.BlockSpec((bn, d), lambda i: (i, 0)),
            pl.BlockSpec((d, d), lambda i: (0, 0)),
            pl.BlockSpec((1, d), lambda i: (0, 0)),
        ],
        out_specs=pl.BlockSpec((bn, d), lambda i: (i, 0)),
        out_shape=jax.ShapeDtypeStruct((n, d), jnp.float32),
    )(x, w, b)


def _tc_scale(degp, xw, bn):
    n, d = xw.shape

    def body(deg_ref, xw_ref, y_ref):
        y_ref[...] = xw_ref[...] * _dinv(deg_ref[...])

    return pl.pallas_call(
        body,
        grid=(n // bn,),
        in_specs=[
            pl.BlockSpec((NC, bn, L), lambda i: (0, i, 0)),
            pl.BlockSpec((bn, d), lambda i: (i, 0)),
        ],
        out_specs=pl.BlockSpec((bn, d), lambda i: (i, 0)),
        out_shape=jax.ShapeDtypeStruct((n, d), jnp.float32),
    )(degp, xw)


def _tc_mid(degp, sp, w, b, n, bn):
    d = sp.shape[-1]

    def body(deg_ref, sp_ref, w_ref, b_ref, o_ref):
        dinv = _dinv(deg_ref[...])
        h = jnp.maximum(sp_ref[0] + sp_ref[1], 0.0) * dinv
        hw = jnp.dot(h, w_ref[...], preferred_element_type=jnp.float32) + b_ref[...]
        o_ref[...] = hw * dinv

    return pl.pallas_call(
        body,
        grid=(n // bn,),
        in_specs=[
            pl.BlockSpec((NC, bn, L), lambda i: (0, i, 0)),
            pl.BlockSpec((NC, bn, d), lambda i: (0, i, 0)),
            pl.BlockSpec((d, d), lambda i: (0, 0)),
            pl.BlockSpec((1, d), lambda i: (0, 0)),
        ],
        out_specs=pl.BlockSpec((bn, d), lambda i: (i, 0)),
        out_shape=jax.ShapeDtypeStruct((n, d), jnp.float32),
    )(degp, sp, w, b)


def _tc_final(degp, sp, n, bn):
    d = sp.shape[-1]

    def body(deg_ref, sp_ref, o_ref):
        o_ref[...] = jnp.maximum(sp_ref[0] + sp_ref[1],
                                 0.0) * _dinv(deg_ref[...])

    return pl.pallas_call(
        body,
        grid=(n // bn,),
        in_specs=[
            pl.BlockSpec((NC, bn, L), lambda i: (0, i, 0)),
            pl.BlockSpec((NC, bn, d), lambda i: (0, i, 0)),
        ],
        out_specs=pl.BlockSpec((bn, d), lambda i: (i, 0)),
        out_shape=jax.ShapeDtypeStruct((n, d), jnp.float32),
    )(degp, sp)


def kernel(x, edge_index, W0, b0, W1, b1):
    n, d = x.shape
    e = edge_index.shape[1]
    loops = jnp.arange(n, dtype=jnp.int32)
    src = jnp.concatenate([edge_index[0].astype(jnp.int32), loops])
    dst = jnp.concatenate([edge_index[1].astype(jnp.int32), loops])
    et = e + n
    nblk = -(-et // (NW * K))
    nblk = -(-nblk // CH) * CH
    pad = NW * nblk * K - et
    n_pad = -(-(n + 1) // (NS * ZR)) * (NS * ZR)
    if pad:
        ar = jnp.arange(pad, dtype=jnp.int32)
        src = jnp.concatenate([src, ar % n])
        dst = jnp.concatenate([dst, n + ar % (n_pad - n)])
    src = src.reshape(NW, nblk // CH, CH, K)
    dst = dst.reshape(NW, nblk // CH, CH, K)

    bn = _pick_bn(n)
    b0r = b0.reshape(1, d)
    b1r = b1.reshape(1, d)

    degp = _make_deg_kernel(n_pad, nblk)(dst)
    agg = _make_agg_kernel(n_pad, nblk, d)
    xw = _tc_matmul(x, W0, b0r, bn)
    y1 = _tc_scale(degp, xw, bn)
    sp1 = agg(y1, src, dst)
    y2 = _tc_mid(degp, sp1, W1, b1r, n, bn)
    sp2 = agg(y2, src, dst)
    return _tc_final(degp, sp2, n, bn)


# speedup vs baseline: 1.0190x; 1.0190x over previous
import functools

import jax
import jax.numpy as jnp
from jax import lax
from jax.experimental import pallas as pl
from jax.experimental.pallas import tpu as pltpu
from jax.experimental.pallas import tpu_sc as plsc

NC = 2
NS = 16
NW = NC * NS
L = 16
K = 72
ZR = 16
NBUF = 4
CH = 9


def _sc_mesh():
    return plsc.VectorSubcoreMesh(
        core_axis_name="c", subcore_axis_name="s",
        num_cores=NC, num_subcores=NS)


def _make_deg_kernel(n_pad, nblk):
    rpt = n_pad // NS

    def body(dst_hbm, out_hbm, idx_v, ones_v, zero_v, acc_sh, sem):
        c = lax.axis_index("c")
        s = lax.axis_index("s")
        wid = c * NS + s
        for i in range(K):
            ones_v[i] = jnp.full((L,), 1.0, jnp.float32)
        for i in range(ZR):
            zero_v[i] = jnp.zeros((L,), jnp.float32)

        @pl.loop(0, rpt // ZR)
        def _zero(r):
            pltpu.sync_copy(zero_v, acc_sh.at[pl.ds(s * rpt + r * ZR, ZR)])

        plsc.subcore_barrier()
        pltpu.sync_copy(dst_hbm.at[wid], idx_v)

        @pl.loop(0, nblk // CH)
        def _scatter(ci):
            for off in range(CH):
                pltpu.async_copy(ones_v, acc_sh.at[idx_v.at[ci, off]], sem,
                                 add=True)

        @pl.loop(0, nblk // CH)
        def _drain(ci):
            for off in range(CH):
                pltpu.make_async_copy(ones_v, acc_sh.at[idx_v.at[0, 0]],
                                      sem).wait()

        plsc.subcore_barrier()
        pltpu.sync_copy(acc_sh.at[pl.ds(s * rpt, rpt)],
                        out_hbm.at[c, pl.ds(s * rpt, rpt)])

    return pl.kernel(
        body,
        out_type=jax.ShapeDtypeStruct((NC, n_pad, L), jnp.float32),
        mesh=_sc_mesh(),
        scratch_types=[
            pltpu.VMEM((nblk // CH, CH, K), jnp.int32),
            pltpu.VMEM((K, L), jnp.float32),
            pltpu.VMEM((ZR, L), jnp.float32),
            pltpu.VMEM_SHARED((n_pad, L), jnp.float32),
            pltpu.SemaphoreType.DMA,
        ])


def _make_agg_kernel(n_pad, nblk, d):
    rpt = n_pad // NS
    nch = nblk // CH

    def body(y_hbm, src_hbm, dst_hbm, out_hbm,
             sidx_v, didx_v, rows_v, acc_sh, gsem, ssem, isem):
        c = lax.axis_index("c")
        s = lax.axis_index("s")
        wid = c * NS + s
        for i in range(ZR):
            for j in range(d // L):
                rows_v[0, i, pl.ds(j * L, L)] = jnp.zeros((L,), jnp.float32)

        @pl.loop(0, rpt // ZR)
        def _zero(r):
            pltpu.sync_copy(rows_v.at[0, pl.ds(0, ZR)],
                            acc_sh.at[pl.ds(s * rpt + r * ZR, ZR)])

        plsc.subcore_barrier()

        idesc = [None] * nch
        gdesc = [None] * nblk
        sdesc = [None] * nblk

        def idx_load(ci):
            ib = ci % 2
            idesc[ci] = (
                pltpu.async_copy(src_hbm.at[wid, ci], sidx_v.at[ib],
                                 isem.at[ib]),
                pltpu.async_copy(dst_hbm.at[wid, ci], didx_v.at[ib],
                                 isem.at[ib]))

        def idx_wait(ci):
            idesc[ci][0].wait()
            idesc[ci][1].wait()

        def gather(blk):
            ci, off = divmod(blk, CH)
            gdesc[blk] = pltpu.async_copy(
                y_hbm.at[sidx_v.at[ci % 2, off]], rows_v.at[blk % NBUF],
                gsem.at[blk % NBUF])

        def scat(blk):
            ci, off = divmod(blk, CH)
            sdesc[blk] = pltpu.async_copy(
                rows_v.at[blk % NBUF], acc_sh.at[didx_v.at[ci % 2, off]],
                ssem.at[blk % NBUF], add=True)

        idx_load(0)
        idx_wait(0)
        if nch > 1:
            idx_load(1)
        waited = {0}
        for blk in range(min(NBUF, nblk)):
            gather(blk)
        for blk in range(nblk):
            gdesc[blk].wait()
            scat(blk)
            nxt = blk + NBUF
            if nxt < nblk:
                sdesc[blk].wait()
                m = blk // CH
                if blk % CH == 1 and m >= 1 and m + 1 < nch:
                    idx_load(m + 1)
                nci = nxt // CH
                if nci not in waited:
                    idx_wait(nci)
                    waited.add(nci)
                gather(nxt)
        for blk in range(max(0, nblk - NBUF), nblk):
            sdesc[blk].wait()
        plsc.subcore_barrier()
        pltpu.sync_copy(acc_sh.at[pl.ds(s * rpt, rpt)],
                        out_hbm.at[c, pl.ds(s * rpt, rpt)])

    return pl.kernel(
        body,
        out_type=jax.ShapeDtypeStruct((NC, n_pad, d), jnp.float32),
        mesh=_sc_mesh(),
        scratch_types=[
            pltpu.VMEM((2, CH, K), jnp.int32),
            pltpu.VMEM((2, CH, K), jnp.int32),
            pltpu.VMEM((NBUF, K, d), jnp.float32),
            pltpu.VMEM_SHARED((n_pad, d), jnp.float32),
            pltpu.SemaphoreType.DMA((NBUF,)),
            pltpu.SemaphoreType.DMA((NBUF,)),
            pltpu.SemaphoreType.DMA((2,)),
        ])


def _dinv(deg_blk):
    return lax.rsqrt(deg_blk[0, :, 0:1] + deg_blk[1, :, 0:1])


def _pick_bn(n):
    for bn in (2048, 2000, 1024, 1000, 512, 500, 256, 250, 128, 125, 64, 50,
               40, 32, 25, 20, 16, 10, 8, 5, 4, 2, 1):
        if n % bn == 0:
            return bn
    return n


def _tc_matmul(x, w, b, bn):
    n, d = x.shape

    def body(x_ref, w_ref, b_ref, y_ref):
        y_ref[...] = jnp.dot(x_ref[...], w_ref[...],
                             preferred_element_type=jnp.float32) + b_ref[...]

    return pl.pallas_call(
        body,
        grid=(n // bn,),
        in_specs=[
            pl.BlockSpec((bn, d), lambda i: (i, 0)),
            pl.BlockSpec((d, d), lambda i: (0, 0)),
            pl.BlockSpec((1, d), lambda i: (0, 0)),
        ],
        out_specs=pl.BlockSpec((bn, d), lambda i: (i, 0)),
        out_shape=jax.ShapeDtypeStruct((n, d), jnp.float32),
    )(x, w, b)


def _tc_scale(degp, xw, bn):
    n, d = xw.shape

    def body(deg_ref, xw_ref, y_ref):
        y_ref[...] = xw_ref[...] * _dinv(deg_ref[...])

    return pl.pallas_call(
        body,
        grid=(n // bn,),
        in_specs=[
            pl.BlockSpec((NC, bn, L), lambda i: (0, i, 0)),
            pl.BlockSpec((bn, d), lambda i: (i, 0)),
        ],
        out_specs=pl.BlockSpec((bn, d), lambda i: (i, 0)),
        out_shape=jax.ShapeDtypeStruct((n, d), jnp.float32),
    )(degp, xw)


def _tc_mid(degp, sp, w, b, n, bn):
    d = sp.shape[-1]

    def body(deg_ref, sp_ref, w_ref, b_ref, o_ref):
        dinv = _dinv(deg_ref[...])
        h = jnp.maximum(sp_ref[0] + sp_ref[1], 0.0) * dinv
        hw = jnp.dot(h, w_ref[...], preferred_element_type=jnp.float32) + b_ref[...]
        o_ref[...] = hw * dinv

    return pl.pallas_call(
        body,
        grid=(n // bn,),
        in_specs=[
            pl.BlockSpec((NC, bn, L), lambda i: (0, i, 0)),
            pl.BlockSpec((NC, bn, d), lambda i: (0, i, 0)),
            pl.BlockSpec((d, d), lambda i: (0, 0)),
            pl.BlockSpec((1, d), lambda i: (0, 0)),
        ],
        out_specs=pl.BlockSpec((bn, d), lambda i: (i, 0)),
        out_shape=jax.ShapeDtypeStruct((n, d), jnp.float32),
    )(degp, sp, w, b)


def _tc_final(degp, sp, n, bn):
    d = sp.shape[-1]

    def body(deg_ref, sp_ref, o_ref):
        o_ref[...] = jnp.maximum(sp_ref[0] + sp_ref[1],
                                 0.0) * _dinv(deg_ref[...])

    return pl.pallas_call(
        body,
        grid=(n // bn,),
        in_specs=[
            pl.BlockSpec((NC, bn, L), lambda i: (0, i, 0)),
            pl.BlockSpec((NC, bn, d), lambda i: (0, i, 0)),
        ],
        out_specs=pl.BlockSpec((bn, d), lambda i: (i, 0)),
        out_shape=jax.ShapeDtypeStruct((n, d), jnp.float32),
    )(degp, sp)


def kernel(x, edge_index, W0, b0, W1, b1):
    n, d = x.shape
    e = edge_index.shape[1]
    loops = jnp.arange(n, dtype=jnp.int32)
    src = jnp.concatenate([edge_index[0].astype(jnp.int32), loops])
    dst = jnp.concatenate([edge_index[1].astype(jnp.int32), loops])
    et = e + n
    nblk = -(-et // (NW * K))
    nblk = -(-nblk // CH) * CH
    pad = NW * nblk * K - et
    n_pad = -(-(n + 1) // (NS * ZR)) * (NS * ZR)
    if pad:
        ar = jnp.arange(pad, dtype=jnp.int32)
        src = jnp.concatenate([src, ar % n])
        dst = jnp.concatenate([dst, n + ar % (n_pad - n)])
    src = src.reshape(NW, nblk // CH, CH, K)
    dst = dst.reshape(NW, nblk // CH, CH, K)

    bn = _pick_bn(n)
    b0r = b0.reshape(1, d)
    b1r = b1.reshape(1, d)

    degp = _make_deg_kernel(n_pad, nblk)(dst)
    agg = _make_agg_kernel(n_pad, nblk, d)
    xw = _tc_matmul(x, W0, b0r, bn)
    y1 = _tc_scale(degp, xw, bn)
    sp1 = agg(y1, src, dst)
    y2 = _tc_mid(degp, sp1, W1, b1r, n, bn)
    sp2 = agg(y2, src, dst)
    return _tc_final(degp, sp2, n, bn)
